# baseline (speedup 1.0000x reference)
import functools

import jax
import jax.numpy as jnp
from jax import lax
from jax.experimental import pallas as pl
from jax.experimental.pallas import tpu as pltpu
from jax.experimental.pallas import tpu_sc as plsc

_B, _S, _D = 4, 8192, 2048
_NPOS = 64
_BS = _B * _S
_NS = 16
_NW = _NS
_ROWS = _B * _NPOS
_RPW = _ROWS // _NW
_WPB = _NPOS // _RPW


@functools.cache
def _sc_scatter():
    @functools.partial(
        pl.kernel,
        mesh=plsc.VectorSubcoreMesh(
            core_axis_name="c", subcore_axis_name="s",
            num_cores=2, num_subcores=8,
        ),
        scratch_types=[
            pltpu.VMEM((_NPOS,), jnp.int32),
            pltpu.VMEM((_RPW, _D), jnp.float32),
            pltpu.SemaphoreType.DMA,
            pltpu.SemaphoreType.DMA,
        ],
    )
    def body(act_hbm, tok_hbm, out_hbm, tok_v, act_v, s_tok, s_act):
        w = lax.axis_index("s") * 2 + lax.axis_index("c")
        g = (w * _RPW) % _NPOS
        st_tok = pltpu.make_async_copy(tok_hbm, tok_v, s_tok)
        st_tok.start()
        st_act = pltpu.make_async_copy(act_hbm.at[pl.ds(g, _RPW)], act_v, s_act)
        st_act.start()
        st_tok.wait()
        st_act.wait()
        row_ids = tok_v[pl.ds(g, _RPW)] + (w // _WPB) * _S
        pltpu.async_copy(act_v, out_hbm.at[row_ids], s_tok).wait()

    return body


def kernel(output, activation, token_position):
    flat = output.reshape(_BS, _D)
    out_ref = jax.new_ref(flat)
    _sc_scatter()(activation, token_position, out_ref)
    return jax.freeze(out_ref).reshape(_B, _S, _D)


# speedup vs baseline: 1.0098x; 1.0098x over previous
import functools

import jax
import jax.numpy as jnp
from jax import lax
from jax.experimental import pallas as pl
from jax.experimental.pallas import tpu as pltpu
from jax.experimental.pallas import tpu_sc as plsc

_B, _S, _D = 4, 8192, 2048
_NPOS = 64
_BS = _B * _S
_NS = 16
_NW = _NS
_ROWS = _B * _NPOS
_RPW = _ROWS // _NW
_WPB = _NPOS // _RPW


@functools.cache
def _sc_scatter():
    @functools.partial(
        pl.kernel,
        mesh=plsc.VectorSubcoreMesh(
            core_axis_name="c", subcore_axis_name="s",
            num_cores=1, num_subcores=_NS,
        ),
        scratch_types=[
            pltpu.VMEM((_NPOS,), jnp.int32),
            pltpu.VMEM((_RPW, _D), jnp.float32),
            pltpu.SemaphoreType.DMA,
            pltpu.SemaphoreType.DMA,
        ],
    )
    def body(act_hbm, tok_hbm, out_hbm, tok_v, act_v, s_tok, s_act):
        w = lax.axis_index("s")
        g = (w * _RPW) % _NPOS
        st_tok = pltpu.make_async_copy(tok_hbm, tok_v, s_tok)
        st_tok.start()
        st_act = pltpu.make_async_copy(act_hbm.at[pl.ds(g, _RPW)], act_v, s_act)
        st_act.start()
        st_tok.wait()
        st_act.wait()
        row_ids = tok_v[pl.ds(g, _RPW)] + (w // _WPB) * _S
        pltpu.async_copy(act_v, out_hbm.at[row_ids], s_tok).wait()

    return body


def kernel(output, activation, token_position):
    flat = output.reshape(_BS, _D)
    out_ref = jax.new_ref(flat)
    _sc_scatter()(activation, token_position, out_ref)
    return jax.freeze(out_ref).reshape(_B, _S, _D)
